# baseline (speedup 1.0000x reference)
import functools

import jax
import jax.numpy as jnp
from jax import lax
from jax.experimental import pallas as pl
from jax.experimental.pallas import tpu as pltpu
from jax.experimental.pallas import tpu_sc as plsc

_NUM_PATCHES = 576
_PROJ_DIM = 768
_NUM_CORES = 2
_NUM_SUBCORES = 16
_CHUNKS = ((0, 192), (192, 192), (384, 176), (560, 16))


def _overlap_body(table_hbm, out_hbm, buf, in_sems, out_sems):
    base0, rows0 = _CHUNKS[0]
    pltpu.make_async_copy(
        table_hbm.at[pl.ds(base0, rows0)],
        buf.at[pl.ds(base0, rows0)],
        in_sems.at[0],
    ).start()
    for k, (base, rows) in enumerate(_CHUNKS):
        pltpu.make_async_copy(
            table_hbm.at[pl.ds(base, rows)],
            buf.at[pl.ds(base, rows)],
            in_sems.at[k],
        ).wait()
        pltpu.make_async_copy(
            buf.at[pl.ds(base, rows)],
            out_hbm.at[pl.ds(base, rows)],
            out_sems.at[k],
        ).start()
        if k + 1 < len(_CHUNKS):
            nbase, nrows = _CHUNKS[k + 1]
            pltpu.make_async_copy(
                table_hbm.at[pl.ds(nbase, nrows)],
                buf.at[pl.ds(nbase, nrows)],
                in_sems.at[k + 1],
            ).start()
    for k, (base, rows) in enumerate(_CHUNKS):
        pltpu.make_async_copy(
            buf.at[pl.ds(base, rows)],
            out_hbm.at[pl.ds(base, rows)],
            out_sems.at[k],
        ).wait()


def kernel(patch, pos_table):
    del patch
    return pl.pallas_call(
        _overlap_body,
        in_specs=[pl.BlockSpec(memory_space=pl.ANY)],
        out_specs=pl.BlockSpec(memory_space=pl.ANY),
        out_shape=jax.ShapeDtypeStruct((_NUM_PATCHES, _PROJ_DIM), jnp.float32),
        scratch_shapes=[
            pltpu.VMEM((_NUM_PATCHES, _PROJ_DIM), jnp.float32),
            pltpu.SemaphoreType.DMA((len(_CHUNKS),)),
            pltpu.SemaphoreType.DMA((len(_CHUNKS),)),
        ],
    )(pos_table)


# speedup vs baseline: 2.1609x; 2.1609x over previous
import functools

import jax
import jax.numpy as jnp
from jax import lax
from jax.experimental import pallas as pl
from jax.experimental.pallas import tpu as pltpu
from jax.experimental.pallas import tpu_sc as plsc

_NUM_PATCHES = 576
_PROJ_DIM = 768
_NUM_CORES = 2
_NUM_SUBCORES = 16
_NCHUNK = 6
_CHUNK_ROWS = _NUM_PATCHES // _NCHUNK


def _overlap_body(table_hbm, out_hbm, buf, in_sems, out_sems):
    for k in range(_NCHUNK):
        pltpu.make_async_copy(
            table_hbm.at[pl.ds(k * _CHUNK_ROWS, _CHUNK_ROWS)],
            buf.at[k],
            in_sems.at[k],
        ).start()
    for k in range(_NCHUNK):
        pltpu.make_async_copy(
            table_hbm.at[pl.ds(k * _CHUNK_ROWS, _CHUNK_ROWS)],
            buf.at[k],
            in_sems.at[k],
        ).wait()
        pltpu.make_async_copy(
            buf.at[k],
            out_hbm.at[pl.ds(k * _CHUNK_ROWS, _CHUNK_ROWS)],
            out_sems.at[k],
        ).start()
    for k in range(_NCHUNK):
        pltpu.make_async_copy(
            buf.at[k],
            out_hbm.at[pl.ds(k * _CHUNK_ROWS, _CHUNK_ROWS)],
            out_sems.at[k],
        ).wait()


def kernel(patch, pos_table):
    del patch
    return pl.pallas_call(
        _overlap_body,
        in_specs=[pl.BlockSpec(memory_space=pl.ANY)],
        out_specs=pl.BlockSpec(memory_space=pl.ANY),
        out_shape=jax.ShapeDtypeStruct((_NUM_PATCHES, _PROJ_DIM), jnp.float32),
        scratch_shapes=[
            pltpu.VMEM((_NCHUNK, _CHUNK_ROWS, _PROJ_DIM), jnp.float32),
            pltpu.SemaphoreType.DMA((_NCHUNK,)),
            pltpu.SemaphoreType.DMA((_NCHUNK,)),
        ],
    )(pos_table)


# speedup vs baseline: 2.1821x; 1.0098x over previous
import functools

import jax
import jax.numpy as jnp
from jax import lax
from jax.experimental import pallas as pl
from jax.experimental.pallas import tpu as pltpu
from jax.experimental.pallas import tpu_sc as plsc

_NUM_PATCHES = 576
_PROJ_DIM = 768
_NUM_CORES = 2
_NUM_SUBCORES = 16
_NCHUNK = 4
_CHUNK_ROWS = _NUM_PATCHES // _NCHUNK


def _overlap_body(table_hbm, out_hbm, buf, in_sems, out_sems):
    for k in range(_NCHUNK):
        pltpu.make_async_copy(
            table_hbm.at[pl.ds(k * _CHUNK_ROWS, _CHUNK_ROWS)],
            buf.at[k],
            in_sems.at[k],
        ).start()
    for k in range(_NCHUNK):
        pltpu.make_async_copy(
            table_hbm.at[pl.ds(k * _CHUNK_ROWS, _CHUNK_ROWS)],
            buf.at[k],
            in_sems.at[k],
        ).wait()
        pltpu.make_async_copy(
            buf.at[k],
            out_hbm.at[pl.ds(k * _CHUNK_ROWS, _CHUNK_ROWS)],
            out_sems.at[k],
        ).start()
    for k in range(_NCHUNK):
        pltpu.make_async_copy(
            buf.at[k],
            out_hbm.at[pl.ds(k * _CHUNK_ROWS, _CHUNK_ROWS)],
            out_sems.at[k],
        ).wait()


def kernel(patch, pos_table):
    del patch
    return pl.pallas_call(
        _overlap_body,
        in_specs=[pl.BlockSpec(memory_space=pl.ANY)],
        out_specs=pl.BlockSpec(memory_space=pl.ANY),
        out_shape=jax.ShapeDtypeStruct((_NUM_PATCHES, _PROJ_DIM), jnp.float32),
        scratch_shapes=[
            pltpu.VMEM((_NCHUNK, _CHUNK_ROWS, _PROJ_DIM), jnp.float32),
            pltpu.SemaphoreType.DMA((_NCHUNK,)),
            pltpu.SemaphoreType.DMA((_NCHUNK,)),
        ],
    )(pos_table)
